# Initial kernel scaffold; baseline (speedup 1.0000x reference)
import jax
import jax.numpy as jnp
from jax.experimental import pallas as pl


def kernel(atomic_numbers, table):
    raise NotImplementedError("write your pallas kernel here")


# speedup vs baseline: 3.2423x; 3.2423x over previous
import functools

import jax
import jax.numpy as jnp
from jax import lax
from jax.experimental import pallas as pl
from jax.experimental.pallas import tpu as pltpu
from jax.experimental.pallas import tpu_sc as plsc

_LANE = 128
_K = 4


@functools.lru_cache(maxsize=None)
def _make_lookup(num_rows: int, depth: int, vocab: int):
    info = plsc.get_sparse_core_info()
    nc, ns = info.num_cores, info.num_subcores
    nw = nc * ns
    assert num_rows % (nw * _K) == 0
    rows_per_w = num_rows // nw
    iters = rows_per_w // _K

    mesh = plsc.VectorSubcoreMesh(core_axis_name="c", subcore_axis_name="s")

    @functools.partial(
        pl.kernel,
        mesh=mesh,
        out_type=jax.ShapeDtypeStruct((num_rows * _LANE, depth), jnp.float32),
        scratch_types=[
            pltpu.VMEM((_K, _LANE), jnp.int32),
            pltpu.VMEM((_K * _LANE, depth), jnp.float32),
            pltpu.SemaphoreType.DMA,
        ],
    )
    def lookup(table_hbm, idx_hbm, out_hbm, idx_v, rows_v, sem):
        wid = lax.axis_index("s") * nc + lax.axis_index("c")
        wbase = wid * rows_per_w

        def body(t, carry):
            g = wbase + t * _K
            pltpu.sync_copy(idx_hbm.at[pl.ds(g, _K)], idx_v)
            cps = [
                pltpu.async_copy(
                    table_hbm.at[idx_v.at[j]],
                    rows_v.at[pl.ds(j * _LANE, _LANE)],
                    sem,
                )
                for j in range(_K)
            ]
            for cp in cps:
                cp.wait()
            pltpu.sync_copy(rows_v, out_hbm.at[pl.ds(g * _LANE, _K * _LANE)])
            return carry

        lax.fori_loop(0, iters, body, 0)

    return lookup


def kernel(atomic_numbers, table):
    b, s = atomic_numbers.shape
    vocab, depth = table.shape
    idx = atomic_numbers.reshape(-1).astype(jnp.int32).reshape(-1, _LANE)
    out = _make_lookup(idx.shape[0], depth, vocab)(table, idx)
    return out.reshape(b, s, depth)


# speedup vs baseline: 11.0825x; 3.4181x over previous
import functools

import jax
import jax.numpy as jnp
from jax import lax
from jax.experimental import pallas as pl
from jax.experimental.pallas import tpu as pltpu
from jax.experimental.pallas import tpu_sc as plsc

_LANE = 128
_K = 4


@functools.lru_cache(maxsize=None)
def _make_lookup(num_rows: int, depth: int, vocab: int):
    info = plsc.get_sparse_core_info()
    nc, ns = info.num_cores, info.num_subcores
    nw = nc * ns
    assert num_rows % (nw * _K) == 0
    rows_per_w = num_rows // nw
    iters = rows_per_w // _K

    mesh = plsc.VectorSubcoreMesh(core_axis_name="c", subcore_axis_name="s")

    @functools.partial(
        pl.kernel,
        mesh=mesh,
        out_type=jax.ShapeDtypeStruct((num_rows * _LANE, depth), jnp.float32),
        scratch_types=[
            pltpu.VMEM((_K, _LANE), jnp.int32),
            pltpu.VMEM((_K * _LANE, depth), jnp.float32),
            pltpu.VMEM_SHARED((vocab, depth), jnp.float32),
            pltpu.SemaphoreType.DMA,
        ],
    )
    def lookup(table_hbm, idx_hbm, out_hbm, idx_v, rows_v, table_sh, sem):
        sid = lax.axis_index("s")
        wid = sid * nc + lax.axis_index("c")
        wbase = wid * rows_per_w

        @pl.when(sid == 0)
        def _():
            pltpu.sync_copy(table_hbm, table_sh)

        plsc.subcore_barrier()

        def body(t, carry):
            g = wbase + t * _K
            pltpu.sync_copy(idx_hbm.at[pl.ds(g, _K)], idx_v)
            cps = [
                pltpu.async_copy(
                    table_sh.at[idx_v.at[j]],
                    rows_v.at[pl.ds(j * _LANE, _LANE)],
                    sem,
                )
                for j in range(_K)
            ]
            for cp in cps:
                cp.wait()
            pltpu.sync_copy(rows_v, out_hbm.at[pl.ds(g * _LANE, _K * _LANE)])
            return carry

        lax.fori_loop(0, iters, body, 0)

    return lookup


def kernel(atomic_numbers, table):
    b, s = atomic_numbers.shape
    vocab, depth = table.shape
    idx = atomic_numbers.reshape(-1).astype(jnp.int32).reshape(-1, _LANE)
    out = _make_lookup(idx.shape[0], depth, vocab)(table, idx)
    return out.reshape(b, s, depth)


# speedup vs baseline: 15.3897x; 1.3887x over previous
import functools

import jax
import jax.numpy as jnp
from jax import lax
from jax.experimental import pallas as pl
from jax.experimental.pallas import tpu as pltpu
from jax.experimental.pallas import tpu_sc as plsc

_LANE = 128
_K = 2


@functools.lru_cache(maxsize=None)
def _make_lookup(num_rows: int, depth: int, vocab: int):
    info = plsc.get_sparse_core_info()
    nc, ns = info.num_cores, info.num_subcores
    nw = nc * ns
    assert num_rows % (nw * _K * 2) == 0
    rows_per_w = num_rows // nw
    iters = rows_per_w // _K
    pairs = iters // 2

    mesh = plsc.VectorSubcoreMesh(core_axis_name="c", subcore_axis_name="s")

    @functools.partial(
        pl.kernel,
        mesh=mesh,
        out_type=jax.ShapeDtypeStruct((num_rows * _LANE, depth), jnp.float32),
        scratch_types=[
            pltpu.VMEM((2, _K, _LANE), jnp.int32),
            pltpu.VMEM((2, _K * _LANE, depth), jnp.float32),
            pltpu.VMEM_SHARED((vocab, depth), jnp.float32),
            pltpu.SemaphoreType.DMA,
            pltpu.SemaphoreType.DMA,
            pltpu.SemaphoreType.DMA,
            pltpu.SemaphoreType.DMA,
        ],
    )
    def lookup(table_hbm, idx_hbm, out_hbm, idx_v, rows_v, table_sh,
               sem_g0, sem_g1, sem_o0, sem_o1):
        sem_g = (sem_g0, sem_g1)
        sem_o = (sem_o0, sem_o1)
        sid = lax.axis_index("s")
        wid = sid * nc + lax.axis_index("c")
        wbase = wid * rows_per_w

        @pl.when(sid == 0)
        def _():
            pltpu.sync_copy(table_hbm, table_sh)

        plsc.subcore_barrier()

        def fire_gather(b, g):
            pltpu.sync_copy(idx_hbm.at[pl.ds(g, _K)], idx_v.at[b])
            for j in range(_K):
                pltpu.async_copy(
                    table_sh.at[idx_v.at[b].at[j]],
                    rows_v.at[b].at[pl.ds(j * _LANE, _LANE)],
                    sem_g[b],
                )

        def drain_gather(b):
            for j in range(_K):
                pltpu.make_async_copy(
                    table_sh.at[idx_v.at[b].at[j]],
                    rows_v.at[b].at[pl.ds(j * _LANE, _LANE)],
                    sem_g[b],
                ).wait()

        def fire_out(b, g):
            pltpu.async_copy(
                rows_v.at[b],
                out_hbm.at[pl.ds(g * _LANE, _K * _LANE)],
                sem_o[b],
            )

        def wait_out(b, g):
            pltpu.make_async_copy(
                rows_v.at[b],
                out_hbm.at[pl.ds(g * _LANE, _K * _LANE)],
                sem_o[b],
            ).wait()

        fire_gather(0, wbase)
        fire_gather(1, wbase + _K)

        def body(u, carry):
            g0 = wbase + 2 * u * _K
            g1 = g0 + _K
            gn0 = g0 + 2 * _K
            gn1 = g1 + 2 * _K
            drain_gather(0)
            fire_out(0, g0)
            drain_gather(1)
            fire_out(1, g1)
            wait_out(0, g0)
            fire_gather(0, gn0)
            wait_out(1, g1)
            fire_gather(1, gn1)
            return carry

        lax.fori_loop(0, pairs - 1, body, 0)

        g0 = wbase + (iters - 2) * _K
        g1 = g0 + _K
        drain_gather(0)
        fire_out(0, g0)
        drain_gather(1)
        fire_out(1, g1)
        wait_out(0, g0)
        wait_out(1, g1)

    return lookup


def kernel(atomic_numbers, table):
    b, s = atomic_numbers.shape
    vocab, depth = table.shape
    idx = atomic_numbers.reshape(-1).astype(jnp.int32).reshape(-1, _LANE)
    out = _make_lookup(idx.shape[0], depth, vocab)(table, idx)
    return out.reshape(b, s, depth)
